# baseline (speedup 1.0000x reference)
import functools

import jax
import jax.numpy as jnp
from jax import lax
from jax.experimental import pallas as pl
from jax.experimental.pallas import tpu as pltpu
from jax.experimental.pallas import tpu_sc as plsc

_NC = 2
_NS = 16
_L = 16
_NW = _NC * _NS
_CROWS = 32
_NBUF = 2
_UNROLL = 4


@functools.lru_cache(maxsize=None)
def _build(nrows: int, ncols: int):
    assert nrows % (_NW * _CROWS * _NBUF) == 0
    rpw = nrows // _NW
    nchunks = rpw // _CROWS
    chunk = _CROWS * ncols
    vecs = chunk // _L
    mesh = plsc.VectorSubcoreMesh(core_axis_name="c", subcore_axis_name="s")

    @functools.partial(
        pl.kernel,
        out_type=jax.ShapeDtypeStruct((_NW, _L), jnp.float32),
        mesh=mesh,
        scratch_types=[
            pltpu.VMEM((_NBUF, _CROWS, ncols), jnp.float32),
            pltpu.VMEM((_NBUF, _CROWS, ncols), jnp.float32),
            pltpu.VMEM((_L,), jnp.float32),
            pltpu.SemaphoreType.DMA,
            pltpu.SemaphoreType.DMA,
            pltpu.SemaphoreType.DMA,
            pltpu.SemaphoreType.DMA,
        ],
    )
    def masked_mse_partials(o_hbm, t_hbm, out_hbm, obuf, tbuf, accv,
                            so0, so1, st0, st1):
        osems = (so0, so1)
        tsems = (st0, st1)
        wid = lax.axis_index("s") * _NC + lax.axis_index("c")
        base = wid * rpw

        def start(ci, b):
            row = pl.multiple_of(base + ci * _CROWS, _CROWS)
            pltpu.async_copy(
                o_hbm.at[pl.ds(row, _CROWS), :], obuf.at[b], osems[b])
            pltpu.async_copy(
                t_hbm.at[pl.ds(row, _CROWS), :], tbuf.at[b], tsems[b])

        def wait(b):
            pltpu.make_async_copy(
                o_hbm.at[pl.ds(0, _CROWS), :], obuf.at[b], osems[b]).wait()
            pltpu.make_async_copy(
                t_hbm.at[pl.ds(0, _CROWS), :], tbuf.at[b], tsems[b]).wait()

        def consume(b, accs):
            vpr = ncols // _L

            def body(r, accs):
                new = list(accs)
                for u in range(vpr):
                    idx = pl.ds(u * _L, _L)
                    o = obuf[b, r, idx]
                    t = tbuf[b, r, idx]
                    d = o - t
                    sq = d * d
                    a = u % _UNROLL
                    new[a] = new[a] + jnp.where(t != 0.0, sq, 0.0)
                return tuple(new)
            return plsc.parallel_loop(
                0, _CROWS, step=1, unroll=2, carry=accs)(body)

        for b in range(_NBUF):
            start(b, b)

        zeros = jnp.zeros((_L,), jnp.float32)
        accs0 = (zeros,) * _UNROLL

        def outer(i, accs):
            for b in range(_NBUF):
                ci = i * _NBUF + b
                wait(b)
                accs = consume(b, accs)

                @pl.when(ci + _NBUF < nchunks)
                def _():
                    start(ci + _NBUF, b)
            return accs

        accs = lax.fori_loop(0, nchunks // _NBUF, outer, accs0)
        total = accs[0] + accs[1] + accs[2] + accs[3]
        accv[...] = total
        pltpu.sync_copy(accv, out_hbm.at[wid])

    return masked_mse_partials


def kernel(output, target):
    nrows, ncols = output.shape
    partials = _build(nrows, ncols)(output, target)
    return jnp.sum(partials) / jnp.float32(output.size)


# speedup vs baseline: 3.7973x; 3.7973x over previous
import functools

import jax
import jax.numpy as jnp
from jax import lax
from jax.experimental import pallas as pl
from jax.experimental.pallas import tpu as pltpu
from jax.experimental.pallas import tpu_sc as plsc

_NC = 2
_NS = 16
_L = 16
_NW = _NC * _NS
_GR = 8
_GPC = 4
_CROWS = _GR * _GPC
_NBUF = 2
_NACC = 8


@functools.lru_cache(maxsize=None)
def _build(nrows: int, ncols: int):
    assert nrows % (_NW * _CROWS * _NBUF) == 0 and ncols % (8 * _L) == 0
    rpw = nrows // _NW
    nchunks = rpw // _CROWS
    vpr = ncols // _L
    mesh = plsc.VectorSubcoreMesh(core_axis_name="c", subcore_axis_name="s")

    @functools.partial(
        pl.kernel,
        out_type=jax.ShapeDtypeStruct((_NW, _L), jnp.float32),
        mesh=mesh,
        scratch_types=[
            pltpu.VMEM((_NBUF, _GPC, _GR, ncols), jnp.float32),
            pltpu.VMEM((_NBUF, _GPC, _GR, ncols), jnp.float32),
            pltpu.VMEM((_L,), jnp.float32),
            pltpu.SemaphoreType.DMA,
            pltpu.SemaphoreType.DMA,
            pltpu.SemaphoreType.DMA,
            pltpu.SemaphoreType.DMA,
        ],
    )
    def masked_mse_partials(o_hbm, t_hbm, out_hbm, obuf, tbuf, accv,
                            so0, so1, st0, st1):
        osems = (so0, so1)
        tsems = (st0, st1)
        wid = lax.axis_index("s") * _NC + lax.axis_index("c")
        base = wid * rpw

        def start(ci, b):
            row = pl.multiple_of(base + ci * _CROWS, _CROWS)
            for g in range(_GPC):
                src = pl.ds(row + g * _GR, _GR)
                pltpu.async_copy(o_hbm.at[src, :], obuf.at[b, g], osems[b])
                pltpu.async_copy(t_hbm.at[src, :], tbuf.at[b, g], tsems[b])

        def wait(b):
            for g in range(_GPC):
                pltpu.make_async_copy(
                    o_hbm.at[pl.ds(0, _GR), :], obuf.at[b, g], osems[b]).wait()
                pltpu.make_async_copy(
                    t_hbm.at[pl.ds(0, _GR), :], tbuf.at[b, g], tsems[b]).wait()

        def consume(b, accs):
            def body(g, accs):
                new = list(accs)
                i = 0
                for dr in range(_GR):
                    for cv in range(vpr):
                        idx = pl.ds(cv * _L, _L)
                        o = obuf[b, g, dr, idx]
                        t = tbuf[b, g, dr, idx]
                        d = o - t
                        sq = d * d
                        a = i % _NACC
                        new[a] = new[a] + jnp.where(t != 0.0, sq, 0.0)
                        i += 1
                return tuple(new)
            return plsc.parallel_loop(0, _GPC, step=1, carry=accs)(body)

        for b in range(_NBUF):
            start(b, b)

        zeros = jnp.zeros((_L,), jnp.float32)
        accs0 = (zeros,) * _NACC

        def outer(i, accs):
            for b in range(_NBUF):
                ci = i * _NBUF + b
                wait(b)
                accs = consume(b, accs)

                @pl.when(ci + _NBUF < nchunks)
                def _():
                    start(ci + _NBUF, b)
            return accs

        accs = lax.fori_loop(0, nchunks // _NBUF, outer, accs0)
        total = accs[0]
        for a in range(1, _NACC):
            total = total + accs[a]
        accv[...] = total
        pltpu.sync_copy(accv, out_hbm.at[wid])

    return masked_mse_partials


def kernel(output, target):
    nrows, ncols = output.shape
    partials = _build(nrows, ncols)(output, target)
    return jnp.sum(partials) / jnp.float32(output.size)
